# baseline (speedup 1.0000x reference)
import functools

import jax
import jax.numpy as jnp
from jax import lax
from jax.experimental import pallas as pl
from jax.experimental.pallas import tpu as pltpu
from jax.experimental.pallas import tpu_sc as plsc

VOCAB = 1000000
DIM = 64
R = 16
B = 16384
L = 20
N = B * L

_info = plsc.get_sparse_core_info()
NC = _info.num_cores
NS = _info.num_subcores
NW = NC * NS
PW = N // NW
CK = 512
NCH = PW // CK
SG = 128
NSG = CK // SG

_mesh = plsc.VectorSubcoreMesh(core_axis_name="c", subcore_axis_name="s")


def _make_gather(width):
    @functools.partial(
        pl.kernel,
        mesh=_mesh,
        out_type=jax.ShapeDtypeStruct((width, N), jnp.float32),
        scratch_types=[
            pltpu.VMEM((CK,), jnp.int32),
            pltpu.VMEM((CK,), jnp.int32),
            pltpu.VMEM((CK, 128), jnp.float32),
            pltpu.VMEM((width, CK), jnp.float32),
            pltpu.SemaphoreType.DMA,
        ],
        compiler_params=pltpu.CompilerParams(needs_layout_passes=False),
    )
    def _gather(view_hbm, idxv_hbm, col_hbm, out_hbm,
                idx_v, col_v, wide_v, tbuf_v, sem):
        wid = lax.axis_index("s") * NC + lax.axis_index("c")
        base = wid * PW

        def chunk_body(c, carry):
            off = base + c * CK
            pltpu.sync_copy(idxv_hbm.at[pl.ds(off, CK)], idx_v)
            pltpu.sync_copy(col_hbm.at[pl.ds(off, CK)], col_v)
            copies = []
            for j in range(NSG):
                isl = idx_v.at[pl.ds(j * SG, SG)]
                dsl = pl.ds(j * SG, SG)
                copies.append(pltpu.async_copy(view_hbm.at[isl], wide_v.at[dsl], sem))
            for cp in copies:
                cp.wait()

            lanes = lax.iota(jnp.int32, 16)

            def tb_body(tb, carry2):
                t0 = tb * 16
                rowv = t0 + lanes
                colbase = col_v[pl.ds(t0, 16)]
                for d in range(width):
                    vals = plsc.load_gather(wide_v, [rowv, colbase + d])
                    tbuf_v[d, pl.ds(t0, 16)] = vals
                return carry2

            lax.fori_loop(0, CK // 16, tb_body, 0)
            pltpu.sync_copy(tbuf_v, out_hbm.at[:, pl.ds(off, CK)])
            return carry

        lax.fori_loop(0, NCH, chunk_body, 0)

    return _gather


_gather_base = _make_gather(DIM)
_gather_u = _make_gather(R)

_BLK = 2048
_NBB = B // _BLK


def _combine_body(ut_ref, bt_ref, wt_ref, o_ref):
    o_ref[...] = (
        bt_ref[...]
        + jnp.dot(wt_ref[...], ut_ref[...], preferred_element_type=jnp.float32)
    )[None]


def kernel(x, base_table, pissa_U, pissa_S, pissa_V):
    idx = x.T.reshape(N)
    bview = base_table.reshape(VOCAB // 2, 128)
    uview = pissa_U.reshape(VOCAB // 8, 128)
    idx_b = lax.shift_right_logical(idx, 1)
    col_b = lax.shift_left((idx & 1), 6)
    idx_u = lax.shift_right_logical(idx, 3)
    col_u = lax.shift_left((idx & 7), 4)
    bt = _gather_base(bview, idx_b, col_b)
    ut = _gather_u(uview, idx_u, col_u)
    wt = (pissa_S[:, None] * pissa_V).T
    out_t = pl.pallas_call(
        _combine_body,
        grid=(L, _NBB),
        in_specs=[
            pl.BlockSpec((R, _BLK), lambda i, j: (0, i * _NBB + j)),
            pl.BlockSpec((DIM, _BLK), lambda i, j: (0, i * _NBB + j)),
            pl.BlockSpec((DIM, R), lambda i, j: (0, 0)),
        ],
        out_specs=pl.BlockSpec((1, DIM, _BLK), lambda i, j: (i, 0, j)),
        out_shape=jax.ShapeDtypeStruct((L, DIM, B), jnp.float32),
    )(ut, bt, wt)
    return out_t.transpose(2, 0, 1)


# speedup vs baseline: 1.0487x; 1.0487x over previous
import functools

import jax
import jax.numpy as jnp
from jax import lax
from jax.experimental import pallas as pl
from jax.experimental.pallas import tpu as pltpu
from jax.experimental.pallas import tpu_sc as plsc

VOCAB = 1000000
DIM = 64
R = 16
B = 16384
L = 20
N = B * L

_info = plsc.get_sparse_core_info()
NC = _info.num_cores
NS = _info.num_subcores
NW = NC * NS
PW = N // NW
CK = 512
NCH = PW // CK
SG = 128
NSG = CK // SG

_mesh = plsc.VectorSubcoreMesh(core_axis_name="c", subcore_axis_name="s")


@functools.partial(
    pl.kernel,
    mesh=_mesh,
    out_type=jax.ShapeDtypeStruct((N, 128), jnp.float32),
    scratch_types=[
        pltpu.VMEM((PW,), jnp.int32),
        pltpu.VMEM((CK, 128), jnp.float32),
        pltpu.SemaphoreType.DMA,
    ],
    compiler_params=pltpu.CompilerParams(needs_layout_passes=False),
)
def _gather_wide(view_hbm, idx_hbm, out_hbm, idx_v, wide_v, sem):
    wid = lax.axis_index("s") * NC + lax.axis_index("c")
    base = wid * PW
    pltpu.sync_copy(idx_hbm.at[pl.ds(base, PW)], idx_v)

    def chunk_body(c, carry):
        off = c * CK
        copies = []
        for j in range(NSG):
            isl = idx_v.at[pl.ds(off + j * SG, SG)]
            dsl = pl.ds(j * SG, SG)
            copies.append(pltpu.async_copy(view_hbm.at[isl], wide_v.at[dsl], sem))
        for cp in copies:
            cp.wait()
        pltpu.sync_copy(wide_v, out_hbm.at[pl.ds(base + off, CK)])
        return carry

    lax.fori_loop(0, NCH, chunk_body, 0)


_BLK = 2048


def _combine_body(u_ref, b_ref, w_ref, o_ref):
    o_ref[...] = b_ref[:, :DIM] + jnp.dot(
        u_ref[:, :R], w_ref[...], preferred_element_type=jnp.float32
    )


def kernel(x, base_table, pissa_U, pissa_S, pissa_V):
    idx = x.reshape(N)
    bpad = jnp.pad(base_table, ((0, 0), (0, 128 - DIM)))
    upad = jnp.pad(pissa_U, ((0, 0), (0, 128 - R)))
    rows_b = _gather_wide(bpad, idx)
    rows_u = _gather_wide(upad, idx)
    w = pissa_S[:, None] * pissa_V
    out = pl.pallas_call(
        _combine_body,
        grid=(N // _BLK,),
        in_specs=[
            pl.BlockSpec((_BLK, 128), lambda i: (i, 0)),
            pl.BlockSpec((_BLK, 128), lambda i: (i, 0)),
            pl.BlockSpec((R, DIM), lambda i: (0, 0)),
        ],
        out_specs=pl.BlockSpec((_BLK, DIM), lambda i: (i, 0)),
        out_shape=jax.ShapeDtypeStruct((N, DIM), jnp.float32),
    )(rows_u, rows_b, w)
    return out.reshape(B, L, DIM)


# speedup vs baseline: 1.2586x; 1.2002x over previous
import functools

import jax
import jax.numpy as jnp
from jax import lax
from jax.experimental import pallas as pl
from jax.experimental.pallas import tpu as pltpu
from jax.experimental.pallas import tpu_sc as plsc

VOCAB = 1000000
DIM = 64
R = 16
B = 16384
L = 20
N = B * L

_info = plsc.get_sparse_core_info()
NC = _info.num_cores
NS = _info.num_subcores
NW = NC * NS
PW = N // NW
CK = 1024
NCH = PW // CK
SG = 128
NSG = CK // SG

_mesh = plsc.VectorSubcoreMesh(core_axis_name="c", subcore_axis_name="s")

_DCH = 320
_NDCH = VOCAB // _DCH
_KMAX = -(-_NDCH // NW)


@functools.partial(
    pl.kernel,
    mesh=_mesh,
    out_type=jax.ShapeDtypeStruct((VOCAB // 8, 128), jnp.float32),
    scratch_types=[
        pltpu.VMEM((_DCH, R), jnp.float32),
        pltpu.VMEM((_DCH // 8, 128), jnp.float32),
    ],
    compiler_params=pltpu.CompilerParams(needs_layout_passes=False),
)
def _u_relayout(u_hbm, out_hbm, in_v, out_v):
    wid = lax.axis_index("s") * NC + lax.axis_index("c")

    def chunk_body(k, carry):
        cid = wid + k * NW

        @pl.when(cid < _NDCH)
        def _():
            r0 = pl.multiple_of(cid * _DCH, _DCH)
            o0 = pl.multiple_of(cid * (_DCH // 8), _DCH // 8)
            pltpu.sync_copy(u_hbm.at[pl.ds(r0, _DCH)], in_v)
            lanes = lax.iota(jnp.int32, 16)

            def g_body(g, carry2):
                grow = jnp.full((16,), g, jnp.int32)
                for s in range(8):
                    vals = plsc.load_gather(
                        in_v, [jnp.full((16,), g * 8 + s, jnp.int32), lanes]
                    )
                    plsc.store_scatter(out_v, [grow, s * R + lanes], vals)
                return carry2

            lax.fori_loop(0, _DCH // 8, g_body, 0)
            pltpu.sync_copy(out_v, out_hbm.at[pl.ds(o0, _DCH // 8)])

        return carry

    lax.fori_loop(0, _KMAX, chunk_body, 0)


def _make_gather(width):
    @functools.partial(
        pl.kernel,
        mesh=_mesh,
        out_type=jax.ShapeDtypeStruct((N, width), jnp.float32),
        scratch_types=[
            pltpu.VMEM((PW,), jnp.int32),
            pltpu.VMEM((CK, width), jnp.float32),
            pltpu.SemaphoreType.DMA,
        ],
        compiler_params=pltpu.CompilerParams(use_tc_tiling_on_sc=False),
    )
    def _gather(table_hbm, idx_hbm, out_hbm, idx_v, rows_v, sem):
        wid = lax.axis_index("s") * NC + lax.axis_index("c")
        base = wid * PW
        pltpu.sync_copy(idx_hbm.at[pl.ds(base, PW)], idx_v)

        def chunk_body(c, carry):
            off = c * CK
            copies = []
            for j in range(NSG):
                isl = idx_v.at[pl.ds(off + j * SG, SG)]
                dsl = pl.ds(j * SG, SG)
                copies.append(pltpu.async_copy(table_hbm.at[isl], rows_v.at[dsl], sem))
            for cp in copies:
                cp.wait()
            pltpu.sync_copy(rows_v, out_hbm.at[pl.ds(base + off, CK)])
            return carry

        lax.fori_loop(0, NCH, chunk_body, 0)

    return _gather


_gather_base = _make_gather(DIM)
_gather_u = _make_gather(R)

_BLK = 2048
_NBB = B // _BLK


def _combine_body(u_ref, b_ref, w_ref, o_ref):
    rows = b_ref[...] + jnp.dot(
        u_ref[...], w_ref[...], preferred_element_type=jnp.float32
    )
    o_ref[...] = rows.T[None]


def kernel(x, base_table, pissa_U, pissa_S, pissa_V):
    idx = x.T.reshape(N)
    u_dense = _u_relayout(pissa_U).reshape(VOCAB, R)
    rows_b = _gather_base(base_table, idx)
    rows_u = _gather_u(u_dense, idx)
    w = pissa_S[:, None] * pissa_V
    out_t = pl.pallas_call(
        _combine_body,
        grid=(L, _NBB),
        in_specs=[
            pl.BlockSpec((_BLK, R), lambda i, j: (i * _NBB + j, 0)),
            pl.BlockSpec((_BLK, DIM), lambda i, j: (i * _NBB + j, 0)),
            pl.BlockSpec((R, DIM), lambda i, j: (0, 0)),
        ],
        out_specs=pl.BlockSpec((1, DIM, _BLK), lambda i, j: (i, 0, j)),
        out_shape=jax.ShapeDtypeStruct((L, DIM, B), jnp.float32),
    )(rows_u, rows_b, w)
    return out_t.transpose(2, 0, 1)


# speedup vs baseline: 1.2589x; 1.0002x over previous
import functools

import jax
import jax.numpy as jnp
from jax import lax
from jax.experimental import pallas as pl
from jax.experimental.pallas import tpu as pltpu
from jax.experimental.pallas import tpu_sc as plsc

VOCAB = 1000000
DIM = 64
R = 16
B = 16384
L = 20
N = B * L

_info = plsc.get_sparse_core_info()
NC = _info.num_cores
NS = _info.num_subcores
NW = NC * NS
PW = N // NW
CK = 1024
NCH = PW // CK
SG = 128
NSG = CK // SG

_mesh = plsc.VectorSubcoreMesh(core_axis_name="c", subcore_axis_name="s")

_DCH = 320
_NDCH = VOCAB // _DCH
_KMAX = -(-_NDCH // NW)


@functools.partial(
    pl.kernel,
    mesh=_mesh,
    out_type=jax.ShapeDtypeStruct((VOCAB // 8, 128), jnp.float32),
    scratch_types=[
        pltpu.VMEM((_DCH, R), jnp.float32),
        pltpu.VMEM((_DCH // 8, 128), jnp.float32),
    ],
    compiler_params=pltpu.CompilerParams(
        needs_layout_passes=False, disable_bounds_checks=True
    ),
)
def _u_relayout(u_hbm, out_hbm, in_v, out_v):
    wid = lax.axis_index("s") * NC + lax.axis_index("c")

    def chunk_body(k, carry):
        cid = wid + k * NW

        @pl.when(cid < _NDCH)
        def _():
            r0 = pl.multiple_of(cid * _DCH, _DCH)
            o0 = pl.multiple_of(cid * (_DCH // 8), _DCH // 8)
            pltpu.sync_copy(u_hbm.at[pl.ds(r0, _DCH)], in_v)
            lanes = lax.iota(jnp.int32, 16)

            def g_body(g, carry2):
                grow = jnp.full((16,), g, jnp.int32)
                for s in range(8):
                    vals = plsc.load_gather(
                        in_v, [jnp.full((16,), g * 8 + s, jnp.int32), lanes]
                    )
                    plsc.store_scatter(out_v, [grow, s * R + lanes], vals)
                return carry2

            lax.fori_loop(0, _DCH // 8, g_body, 0)
            pltpu.sync_copy(out_v, out_hbm.at[pl.ds(o0, _DCH // 8)])

        return carry

    lax.fori_loop(0, _KMAX, chunk_body, 0)


def _make_gather(width):
    @functools.partial(
        pl.kernel,
        mesh=_mesh,
        out_type=jax.ShapeDtypeStruct((N, width), jnp.float32),
        scratch_types=[
            pltpu.VMEM((PW,), jnp.int32),
            pltpu.VMEM((CK, width), jnp.float32),
            pltpu.SemaphoreType.DMA,
        ],
        compiler_params=pltpu.CompilerParams(
            use_tc_tiling_on_sc=False, disable_bounds_checks=True
        ),
    )
    def _gather(table_hbm, idx_hbm, out_hbm, idx_v, rows_v, sem):
        wid = lax.axis_index("s") * NC + lax.axis_index("c")
        base = wid * PW
        pltpu.sync_copy(idx_hbm.at[pl.ds(base, PW)], idx_v)

        def chunk_body(c, carry):
            off = c * CK
            copies = []
            for j in range(NSG):
                isl = idx_v.at[pl.ds(off + j * SG, SG)]
                dsl = pl.ds(j * SG, SG)
                copies.append(pltpu.async_copy(table_hbm.at[isl], rows_v.at[dsl], sem))
            for cp in copies:
                cp.wait()
            pltpu.sync_copy(rows_v, out_hbm.at[pl.ds(base + off, CK)])
            return carry

        lax.fori_loop(0, NCH, chunk_body, 0)

    return _gather


_gather_base = _make_gather(DIM)
_gather_u = _make_gather(R)

_BLK = 2048
_NBB = B // _BLK


def _combine_body(u_ref, b_ref, w_ref, o_ref):
    rows = b_ref[...] + jnp.dot(
        u_ref[...], w_ref[...], preferred_element_type=jnp.float32
    )
    o_ref[...] = rows.T[None]


def kernel(x, base_table, pissa_U, pissa_S, pissa_V):
    idx = x.T.reshape(N)
    u_dense = _u_relayout(pissa_U).reshape(VOCAB, R)
    rows_b = _gather_base(base_table, idx)
    rows_u = _gather_u(u_dense, idx)
    w = pissa_S[:, None] * pissa_V
    out_t = pl.pallas_call(
        _combine_body,
        grid=(L, _NBB),
        in_specs=[
            pl.BlockSpec((_BLK, R), lambda i, j: (i * _NBB + j, 0)),
            pl.BlockSpec((_BLK, DIM), lambda i, j: (i * _NBB + j, 0)),
            pl.BlockSpec((R, DIM), lambda i, j: (0, 0)),
        ],
        out_specs=pl.BlockSpec((1, DIM, _BLK), lambda i, j: (i, 0, j)),
        out_shape=jax.ShapeDtypeStruct((L, DIM, B), jnp.float32),
    )(rows_u, rows_b, w)
    return out_t.transpose(2, 0, 1)


# speedup vs baseline: 1.4239x; 1.1311x over previous
import functools

import jax
import jax.numpy as jnp
from jax import lax
from jax.experimental import pallas as pl
from jax.experimental.pallas import tpu as pltpu
from jax.experimental.pallas import tpu_sc as plsc

VOCAB = 1000000
DIM = 64
R = 16
B = 16384
L = 20
N = B * L

_info = plsc.get_sparse_core_info()
NC = _info.num_cores
NS = _info.num_subcores
NW = NC * NS
PW = N // NW
CK = 1024
NCH = PW // CK
SG = 128
NSG = CK // SG

_mesh = plsc.VectorSubcoreMesh(core_axis_name="c", subcore_axis_name="s")

_VP = 1000064
_DCH = 1664
_NDCH = _VP // _DCH
_KMAX = -(-_NDCH // NW)


@functools.partial(
    pl.kernel,
    mesh=_mesh,
    out_type=jax.ShapeDtypeStruct((_VP // 8, 128), jnp.float32),
    scratch_types=[
        pltpu.VMEM((R, _DCH), jnp.float32),
        pltpu.VMEM((_DCH // 8, 128), jnp.float32),
    ],
    compiler_params=pltpu.CompilerParams(
        needs_layout_passes=False, disable_bounds_checks=True
    ),
)
def _u_relayout(ut_hbm, out_hbm, in_v, out_v):
    wid = lax.axis_index("s") * NC + lax.axis_index("c")
    lanes = lax.iota(jnp.int32, 16)

    def chunk_body(k, carry):
        cid = wid + k * NW

        @pl.when(cid < _NDCH)
        def _():
            c0 = pl.multiple_of(cid * _DCH, _DCH)
            o0 = pl.multiple_of(cid * (_DCH // 8), _DCH // 8)
            pltpu.sync_copy(ut_hbm.at[:, pl.ds(c0, _DCH)], in_v)

            def g_body(g, carry2):
                grow = jnp.full((16,), g, jnp.int32)
                for s in range(8):
                    vals = plsc.load_gather(
                        in_v, [lanes, jnp.full((16,), g * 8 + s, jnp.int32)]
                    )
                    plsc.store_scatter(out_v, [grow, s * R + lanes], vals)
                return carry2

            lax.fori_loop(0, _DCH // 8, g_body, 0)
            pltpu.sync_copy(out_v, out_hbm.at[pl.ds(o0, _DCH // 8)])

        return carry

    lax.fori_loop(0, _KMAX, chunk_body, 0)


def _make_gather(width):
    @functools.partial(
        pl.kernel,
        mesh=_mesh,
        out_type=jax.ShapeDtypeStruct((N, width), jnp.float32),
        scratch_types=[
            pltpu.VMEM((PW,), jnp.int32),
            pltpu.VMEM((CK, width), jnp.float32),
            pltpu.SemaphoreType.DMA,
        ],
        compiler_params=pltpu.CompilerParams(
            use_tc_tiling_on_sc=False, disable_bounds_checks=True
        ),
    )
    def _gather(table_hbm, idx_hbm, out_hbm, idx_v, rows_v, sem):
        wid = lax.axis_index("s") * NC + lax.axis_index("c")
        base = wid * PW
        pltpu.sync_copy(idx_hbm.at[pl.ds(base, PW)], idx_v)

        def chunk_body(c, carry):
            off = c * CK
            copies = []
            for j in range(NSG):
                isl = idx_v.at[pl.ds(off + j * SG, SG)]
                dsl = pl.ds(j * SG, SG)
                copies.append(pltpu.async_copy(table_hbm.at[isl], rows_v.at[dsl], sem))
            for cp in copies:
                cp.wait()
            pltpu.sync_copy(rows_v, out_hbm.at[pl.ds(base + off, CK)])
            return carry

        lax.fori_loop(0, NCH, chunk_body, 0)

    return _gather


_gather_base = _make_gather(DIM)
_gather_u = _make_gather(R)

_BLK = 2048
_NBB = B // _BLK


def _combine_body(u_ref, b_ref, w_ref, o_ref):
    rows = b_ref[...] + jnp.dot(
        u_ref[...], w_ref[...], preferred_element_type=jnp.float32
    )
    o_ref[...] = rows.T[None]


def kernel(x, base_table, pissa_U, pissa_S, pissa_V):
    idx = x.T.reshape(N)
    ut = jnp.pad(pissa_U.T, ((0, 0), (0, _VP - VOCAB)))
    u_dense = _u_relayout(ut).reshape(_VP, R)
    rows_b = _gather_base(base_table, idx)
    rows_u = _gather_u(u_dense, idx)
    w = pissa_S[:, None] * pissa_V
    out_t = pl.pallas_call(
        _combine_body,
        grid=(L, _NBB),
        in_specs=[
            pl.BlockSpec((_BLK, R), lambda i, j: (i * _NBB + j, 0)),
            pl.BlockSpec((_BLK, DIM), lambda i, j: (i * _NBB + j, 0)),
            pl.BlockSpec((R, DIM), lambda i, j: (0, 0)),
        ],
        out_specs=pl.BlockSpec((1, DIM, _BLK), lambda i, j: (i, 0, j)),
        out_shape=jax.ShapeDtypeStruct((L, DIM, B), jnp.float32),
    )(rows_u, rows_b, w)
    return out_t.transpose(2, 0, 1)


# speedup vs baseline: 1.6987x; 1.1930x over previous
import functools

import jax
import jax.numpy as jnp
from jax import lax
from jax.experimental import pallas as pl
from jax.experimental.pallas import tpu as pltpu
from jax.experimental.pallas import tpu_sc as plsc

VOCAB = 1000000
DIM = 64
R = 16
B = 16384
L = 20
N = B * L

_info = plsc.get_sparse_core_info()
NC = _info.num_cores
NS = _info.num_subcores
NW = NC * NS
PW = N // NW
CK = 1024
NCH = PW // CK
SG = 128
NSG = CK // SG

_mesh = plsc.VectorSubcoreMesh(core_axis_name="c", subcore_axis_name="s")

_VP = 1000064
_DCH = 1664
_NDCH = _VP // _DCH
_KMAX = -(-_NDCH // NW)


@functools.partial(
    pl.kernel,
    mesh=_mesh,
    out_type=jax.ShapeDtypeStruct((_VP // 8, 128), jnp.float32),
    scratch_types=[
        pltpu.VMEM((R, _DCH), jnp.float32),
        pltpu.VMEM((_DCH // 8, 128), jnp.float32),
    ],
    compiler_params=pltpu.CompilerParams(
        needs_layout_passes=False, disable_bounds_checks=True
    ),
)
def _u_relayout(ut_hbm, out_hbm, in_v, out_v):
    wid = lax.axis_index("s") * NC + lax.axis_index("c")
    lanes = lax.iota(jnp.int32, 16)

    def chunk_body(k, carry):
        cid = wid + k * NW

        @pl.when(cid < _NDCH)
        def _():
            c0 = pl.multiple_of(cid * _DCH, _DCH)
            o0 = pl.multiple_of(cid * (_DCH // 8), _DCH // 8)
            pltpu.sync_copy(ut_hbm.at[:, pl.ds(c0, _DCH)], in_v)

            def g_body(g, carry2):
                grow = jnp.full((16,), g, jnp.int32)
                for s in range(8):
                    vals = plsc.load_gather(
                        in_v, [lanes, jnp.full((16,), g * 8 + s, jnp.int32)]
                    )
                    plsc.store_scatter(out_v, [grow, s * R + lanes], vals)
                return carry2

            lax.fori_loop(0, _DCH // 8, g_body, 0)
            pltpu.sync_copy(out_v, out_hbm.at[pl.ds(o0, _DCH // 8)])

        return carry

    lax.fori_loop(0, _KMAX, chunk_body, 0)


def _make_gather(width):
    @functools.partial(
        pl.kernel,
        mesh=_mesh,
        out_type=jax.ShapeDtypeStruct((N, 128), jnp.float32),
        scratch_types=[
            pltpu.VMEM((PW,), jnp.int32),
            pltpu.VMEM((CK, width), jnp.float32),
            pltpu.SemaphoreType.DMA,
        ],
        compiler_params=pltpu.CompilerParams(
            use_tc_tiling_on_sc=False, disable_bounds_checks=True
        ),
    )
    def _gather(table_hbm, idx_hbm, out_hbm, idx_v, rows_v, sem):
        wid = lax.axis_index("s") * NC + lax.axis_index("c")
        base = wid * PW
        pltpu.sync_copy(idx_hbm.at[pl.ds(base, PW)], idx_v)

        def chunk_body(c, carry):
            off = c * CK
            copies = []
            for j in range(NSG):
                isl = idx_v.at[pl.ds(off + j * SG, SG)]
                dsl = pl.ds(j * SG, SG)
                copies.append(pltpu.async_copy(table_hbm.at[isl], rows_v.at[dsl], sem))
            for cp in copies:
                cp.wait()
            pltpu.sync_copy(
                rows_v, out_hbm.at[pl.ds(base + off, CK), pl.ds(0, width)]
            )
            return carry

        lax.fori_loop(0, NCH, chunk_body, 0)

    return _gather


_gather_base = _make_gather(DIM)
_gather_u = _make_gather(R)

_BLK = 2048
_NBB = B // _BLK


def _combine_body(u_ref, b_ref, w_ref, o_ref):
    rows = b_ref[:, :DIM] + jnp.dot(
        u_ref[:, :R], w_ref[...], preferred_element_type=jnp.float32
    )
    o_ref[...] = rows.T[None]


def kernel(x, base_table, pissa_U, pissa_S, pissa_V):
    idx = x.T.reshape(N)
    ut = jnp.pad(pissa_U.T, ((0, 0), (0, _VP - VOCAB)))
    u_dense = _u_relayout(ut).reshape(_VP, R)
    rows_b = _gather_base(base_table, idx)
    rows_u = _gather_u(u_dense, idx)
    w = pissa_S[:, None] * pissa_V
    out_t = pl.pallas_call(
        _combine_body,
        grid=(L, _NBB),
        in_specs=[
            pl.BlockSpec((_BLK, 128), lambda i, j: (i * _NBB + j, 0)),
            pl.BlockSpec((_BLK, 128), lambda i, j: (i * _NBB + j, 0)),
            pl.BlockSpec((R, DIM), lambda i, j: (0, 0)),
        ],
        out_specs=pl.BlockSpec((1, DIM, _BLK), lambda i, j: (i, 0, j)),
        out_shape=jax.ShapeDtypeStruct((L, DIM, B), jnp.float32),
    )(rows_u, rows_b, w)
    return out_t.transpose(2, 0, 1)


# speedup vs baseline: 1.7016x; 1.0017x over previous
import functools

import jax
import jax.numpy as jnp
from jax import lax
from jax.experimental import pallas as pl
from jax.experimental.pallas import tpu as pltpu
from jax.experimental.pallas import tpu_sc as plsc

VOCAB = 1000000
DIM = 64
R = 16
B = 16384
L = 20
N = B * L

_info = plsc.get_sparse_core_info()
NC = _info.num_cores
NS = _info.num_subcores
NW = NC * NS
PW = N // NW
CK = 1024
NCH = PW // CK
SG = 128
NSG = CK // SG

_mesh = plsc.VectorSubcoreMesh(core_axis_name="c", subcore_axis_name="s")

_VP = 1000064
_DCH = 1664
_NDCH = _VP // _DCH
_KMAX = -(-_NDCH // NW)


@functools.partial(
    pl.kernel,
    mesh=_mesh,
    out_type=jax.ShapeDtypeStruct((_VP // 8, 128), jnp.float32),
    scratch_types=[
        pltpu.VMEM((R, _DCH), jnp.float32),
        pltpu.VMEM((_DCH // 8, 128), jnp.float32),
    ],
    compiler_params=pltpu.CompilerParams(
        needs_layout_passes=False, disable_bounds_checks=True
    ),
)
def _u_relayout(ut_hbm, out_hbm, in_v, out_v):
    wid = lax.axis_index("s") * NC + lax.axis_index("c")
    lanes = lax.iota(jnp.int32, 16)

    def chunk_body(k, carry):
        cid = wid + k * NW

        @pl.when(cid < _NDCH)
        def _():
            c0 = pl.multiple_of(cid * _DCH, _DCH)
            o0 = pl.multiple_of(cid * (_DCH // 8), _DCH // 8)
            pltpu.sync_copy(ut_hbm.at[:, pl.ds(c0, _DCH)], in_v)

            def g_body(g, carry2):
                for s in range(8):
                    vals = plsc.load_gather(
                        in_v, [lanes, jnp.full((16,), g * 8 + s, jnp.int32)]
                    )
                    out_v[g, pl.ds(s * R, R)] = vals
                return carry2

            lax.fori_loop(0, _DCH // 8, g_body, 0)
            pltpu.sync_copy(out_v, out_hbm.at[pl.ds(o0, _DCH // 8)])

        return carry

    lax.fori_loop(0, _KMAX, chunk_body, 0)


def _make_gather(width):
    @functools.partial(
        pl.kernel,
        mesh=_mesh,
        out_type=jax.ShapeDtypeStruct((N, 128), jnp.float32),
        scratch_types=[
            pltpu.VMEM((PW,), jnp.int32),
            pltpu.VMEM((CK, width), jnp.float32),
            pltpu.SemaphoreType.DMA,
        ],
        compiler_params=pltpu.CompilerParams(
            use_tc_tiling_on_sc=False, disable_bounds_checks=True
        ),
    )
    def _gather(table_hbm, idx_hbm, out_hbm, idx_v, rows_v, sem):
        wid = lax.axis_index("s") * NC + lax.axis_index("c")
        base = wid * PW
        pltpu.sync_copy(idx_hbm.at[pl.ds(base, PW)], idx_v)

        def chunk_body(c, carry):
            off = c * CK
            copies = []
            for j in range(NSG):
                isl = idx_v.at[pl.ds(off + j * SG, SG)]
                dsl = pl.ds(j * SG, SG)
                copies.append(pltpu.async_copy(table_hbm.at[isl], rows_v.at[dsl], sem))
            for cp in copies:
                cp.wait()
            pltpu.sync_copy(
                rows_v, out_hbm.at[pl.ds(base + off, CK), pl.ds(0, width)]
            )
            return carry

        lax.fori_loop(0, NCH, chunk_body, 0)

    return _gather


_gather_base = _make_gather(DIM)
_gather_u = _make_gather(R)

_BLK = 2048
_NBB = B // _BLK


def _combine_body(u_ref, b_ref, w_ref, o_ref):
    rows = b_ref[:, :DIM] + jnp.dot(
        u_ref[:, :R], w_ref[...], preferred_element_type=jnp.float32
    )
    o_ref[...] = rows.T[None]


def kernel(x, base_table, pissa_U, pissa_S, pissa_V):
    idx = x.T.reshape(N)
    ut = jnp.pad(pissa_U.T, ((0, 0), (0, _VP - VOCAB)))
    u_dense = _u_relayout(ut).reshape(_VP, R)
    rows_b = _gather_base(base_table, idx)
    rows_u = _gather_u(u_dense, idx)
    w = pissa_S[:, None] * pissa_V
    out_t = pl.pallas_call(
        _combine_body,
        grid=(L, _NBB),
        in_specs=[
            pl.BlockSpec((_BLK, 128), lambda i, j: (i * _NBB + j, 0)),
            pl.BlockSpec((_BLK, 128), lambda i, j: (i * _NBB + j, 0)),
            pl.BlockSpec((R, DIM), lambda i, j: (0, 0)),
        ],
        out_specs=pl.BlockSpec((1, DIM, _BLK), lambda i, j: (i, 0, j)),
        out_shape=jax.ShapeDtypeStruct((L, DIM, B), jnp.float32),
    )(rows_u, rows_b, w)
    return out_t.transpose(2, 0, 1)


# speedup vs baseline: 1.7022x; 1.0003x over previous
import functools

import jax
import jax.numpy as jnp
from jax import lax
from jax.experimental import pallas as pl
from jax.experimental.pallas import tpu as pltpu
from jax.experimental.pallas import tpu_sc as plsc

VOCAB = 1000000
DIM = 64
R = 16
B = 16384
L = 20
N = B * L

_info = plsc.get_sparse_core_info()
NC = _info.num_cores
NS = _info.num_subcores
NW = NC * NS
PW = N // NW
CK = 1024
NCH = PW // CK
SG = 128
NSG = CK // SG

_mesh = plsc.VectorSubcoreMesh(core_axis_name="c", subcore_axis_name="s")

_VP = 1000064
_DCH = 1664
_NDCH = _VP // _DCH
_KMAX = -(-_NDCH // NW)


@functools.partial(
    pl.kernel,
    mesh=_mesh,
    out_type=jax.ShapeDtypeStruct((_VP // 8, 128), jnp.float32),
    scratch_types=[
        pltpu.VMEM((R, _DCH + 8), jnp.float32),
        pltpu.VMEM((_DCH // 8, 128), jnp.float32),
    ],
    compiler_params=pltpu.CompilerParams(
        needs_layout_passes=False, disable_bounds_checks=True
    ),
)
def _u_relayout(ut_hbm, out_hbm, in_v, out_v):
    wid = lax.axis_index("s") * NC + lax.axis_index("c")
    lanes = lax.iota(jnp.int32, 16)

    def chunk_body(k, carry):
        cid = wid + k * NW

        @pl.when(cid < _NDCH)
        def _():
            c0 = pl.multiple_of(cid * _DCH, _DCH)
            o0 = pl.multiple_of(cid * (_DCH // 8), _DCH // 8)
            pltpu.sync_copy(ut_hbm.at[:, pl.ds(c0, _DCH)], in_v.at[:, pl.ds(0, _DCH)])

            def g_body(g, carry2):
                for s in range(8):
                    vals = plsc.load_gather(
                        in_v, [lanes, jnp.full((16,), g * 8 + s, jnp.int32)]
                    )
                    out_v[g, pl.ds(s * R, R)] = vals
                return carry2

            lax.fori_loop(0, _DCH // 8, g_body, 0)
            pltpu.sync_copy(out_v, out_hbm.at[pl.ds(o0, _DCH // 8)])

        return carry

    lax.fori_loop(0, _KMAX, chunk_body, 0)


def _make_gather(width):
    @functools.partial(
        pl.kernel,
        mesh=_mesh,
        out_type=jax.ShapeDtypeStruct((N, 128), jnp.float32),
        scratch_types=[
            pltpu.VMEM((PW,), jnp.int32),
            pltpu.VMEM((CK, width), jnp.float32),
            pltpu.SemaphoreType.DMA,
        ],
        compiler_params=pltpu.CompilerParams(
            use_tc_tiling_on_sc=False, disable_bounds_checks=True
        ),
    )
    def _gather(table_hbm, idx_hbm, out_hbm, idx_v, rows_v, sem):
        wid = lax.axis_index("s") * NC + lax.axis_index("c")
        base = wid * PW
        pltpu.sync_copy(idx_hbm.at[pl.ds(base, PW)], idx_v)

        def chunk_body(c, carry):
            off = c * CK
            copies = []
            for j in range(NSG):
                isl = idx_v.at[pl.ds(off + j * SG, SG)]
                dsl = pl.ds(j * SG, SG)
                copies.append(pltpu.async_copy(table_hbm.at[isl], rows_v.at[dsl], sem))
            for cp in copies:
                cp.wait()
            pltpu.sync_copy(
                rows_v, out_hbm.at[pl.ds(base + off, CK), pl.ds(0, width)]
            )
            return carry

        lax.fori_loop(0, NCH, chunk_body, 0)

    return _gather


_gather_base = _make_gather(DIM)
_gather_u = _make_gather(R)

_BLK = 2048
_NBB = B // _BLK


def _combine_body(u_ref, b_ref, w_ref, o_ref):
    rows = b_ref[:, :DIM] + jnp.dot(
        u_ref[:, :R], w_ref[...], preferred_element_type=jnp.float32
    )
    o_ref[...] = rows.T[None]


def kernel(x, base_table, pissa_U, pissa_S, pissa_V):
    idx = x.T.reshape(N)
    ut = jnp.pad(pissa_U.T, ((0, 0), (0, _VP - VOCAB)))
    u_dense = _u_relayout(ut).reshape(_VP, R)
    rows_b = _gather_base(base_table, idx)
    rows_u = _gather_u(u_dense, idx)
    w = pissa_S[:, None] * pissa_V
    out_t = pl.pallas_call(
        _combine_body,
        grid=(L, _NBB),
        in_specs=[
            pl.BlockSpec((_BLK, 128), lambda i, j: (i * _NBB + j, 0)),
            pl.BlockSpec((_BLK, 128), lambda i, j: (i * _NBB + j, 0)),
            pl.BlockSpec((R, DIM), lambda i, j: (0, 0)),
        ],
        out_specs=pl.BlockSpec((1, DIM, _BLK), lambda i, j: (i, 0, j)),
        out_shape=jax.ShapeDtypeStruct((L, DIM, B), jnp.float32),
    )(rows_u, rows_b, w)
    return out_t.transpose(2, 0, 1)
